# baseline (speedup 1.0000x reference)
import functools

import jax
import jax.numpy as jnp
from jax import lax
from jax.experimental import pallas as pl
from jax.experimental.pallas import tpu as pltpu
from jax.experimental.pallas import tpu_sc as plsc

_N = 10000
_E = 320000
_H = 128
_G = 64
_LAYERS = 3

_CHUNK = 3200
_NCH = _E // _CHUNK
_FPW = 4


def _sc_message_pass(h_t, e_t, src, dst):
    mesh = plsc.VectorSubcoreMesh(core_axis_name="c", subcore_axis_name="s")

    @functools.partial(
        pl.kernel,
        mesh=mesh,
        out_type=jax.ShapeDtypeStruct((_H, _N), jnp.float32),
        scratch_types=(
            [pltpu.VMEM((_N,), jnp.float32) for _ in range(_FPW)]
            + [pltpu.VMEM((_N,), jnp.float32) for _ in range(_FPW)]
            + [
                pltpu.VMEM((2, _CHUNK), jnp.int32),
                pltpu.VMEM((2, _CHUNK), jnp.int32),
                pltpu.VMEM((2, _FPW, _CHUNK), jnp.float32),
            ]
            + [pltpu.SemaphoreType.DMA for _ in range(6)]
        ),
        compiler_params=pltpu.CompilerParams(needs_layout_passes=False),
    )
    def mp_kernel(h_hbm, e_hbm, src_hbm, dst_hbm, agg_hbm,
                  h0, h1, h2, h3, a0, a1, a2, a3,
                  src_v, dst_v, e_v,
                  ss0, ss1, sd0, sd1, se0, se1):
        h_f = [h0, h1, h2, h3]
        a_f = [a0, a1, a2, a3]
        sem_s = [ss0, ss1]
        sem_d = [sd0, sd1]
        sem_e = [se0, se1]

        wid = lax.axis_index("s") * 2 + lax.axis_index("c")
        f0 = wid * _FPW

        for f in range(_FPW):
            pltpu.sync_copy(h_hbm.at[f0 + f], h_f[f])

        def zero_body(i, _):
            ii = pl.multiple_of(i * 16, 16)
            z = jnp.zeros((16,), jnp.float32)
            for f in range(_FPW):
                a_f[f][pl.ds(ii, 16)] = z
            return _

        lax.fori_loop(0, _N // 16, zero_body, None)

        def start_chunk(c, b):
            off = pl.multiple_of(c * _CHUNK, _CHUNK)
            pltpu.async_copy(src_hbm.at[pl.ds(off, _CHUNK)], src_v.at[b],
                             sem_s[b])
            pltpu.async_copy(dst_hbm.at[pl.ds(off, _CHUNK)], dst_v.at[b],
                             sem_d[b])
            pltpu.async_copy(e_hbm.at[pl.ds(f0, _FPW), pl.ds(off, _CHUNK)],
                             e_v.at[b], sem_e[b])

        def wait_chunk(b):
            pltpu.make_async_copy(src_hbm.at[pl.ds(0, _CHUNK)], src_v.at[b],
                                  sem_s[b]).wait()
            pltpu.make_async_copy(dst_hbm.at[pl.ds(0, _CHUNK)], dst_v.at[b],
                                  sem_d[b]).wait()
            pltpu.make_async_copy(e_hbm.at[pl.ds(0, _FPW), pl.ds(0, _CHUNK)],
                                  e_v.at[b], sem_e[b]).wait()

        start_chunk(0, 0)
        start_chunk(1, 1)

        def edge_body(j, b):
            jj = pl.multiple_of(j * 16, 16)
            sv = src_v[b, pl.ds(jj, 16)]
            dv = dst_v[b, pl.ds(jj, 16)]
            for f in range(_FPW):
                hv = plsc.load_gather(h_f[f], [sv])
                ev = e_v[b, f, pl.ds(jj, 16)]
                m = jnp.maximum(hv + ev, 0.0)
                plsc.addupdate_scatter(a_f[f], [dv], m)

        def outer_body(g, _):
            for b in range(2):
                c = g * 2 + b
                wait_chunk(b)

                @plsc.parallel_loop(0, _CHUNK // 16, unroll=8)
                def _(j, _b=b):
                    edge_body(j, _b)

                @pl.when(c + 2 < _NCH)
                def _():
                    start_chunk(c + 2, b)
            return _

        lax.fori_loop(0, _NCH // 2, outer_body, None)

        for f in range(_FPW):
            pltpu.sync_copy(a_f[f], agg_hbm.at[f0 + f])

    return mp_kernel(h_t, e_t, src, dst)


def _node_enc_body(x_ref, w_ref, b_ref, out_ref):
    out_ref[...] = lax.dot_general(
        w_ref[...], x_ref[...], (((0,), (1,)), ((), ())),
        preferred_element_type=jnp.float32) + b_ref[...]


def _node_enc(x, w, b):
    return pl.pallas_call(
        _node_enc_body,
        out_shape=jax.ShapeDtypeStruct((_H, _N), jnp.float32),
    )(x, w, b.reshape(_H, 1))


def _edge_enc_body(attr_ref, w_ref, b_ref, out_hbm, scratch, sem, *, block):
    i = pl.program_id(0)
    scratch[...] = lax.dot_general(
        w_ref[...], attr_ref[...], (((0,), (1,)), ((), ())),
        preferred_element_type=jnp.float32) + b_ref[...]
    pltpu.async_copy(scratch, out_hbm.at[:, pl.ds(i * block, block)],
                     sem).wait()


def _edge_enc(attr, w, b, block=16000):
    return pl.pallas_call(
        functools.partial(_edge_enc_body, block=block),
        grid=(_E // block,),
        in_specs=[
            pl.BlockSpec((block, 16), lambda i: (i, 0)),
            pl.BlockSpec((16, _H), lambda i: (0, 0)),
            pl.BlockSpec((_H, 1), lambda i: (0, 0)),
        ],
        out_specs=pl.BlockSpec(memory_space=pl.ANY),
        scratch_shapes=[pltpu.VMEM((_H, block), jnp.float32),
                        pltpu.SemaphoreType.DMA],
        out_shape=jax.ShapeDtypeStruct((_H, _E), jnp.float32),
    )(attr, w, b.reshape(_H, 1))


def _mlp_body(h_ref, agg_ref, w1_ref, b1_ref, g_ref, bb_ref, w2_ref, b2_ref,
              out_ref):
    t = h_ref[...] + agg_ref[...]
    t = lax.dot_general(w1_ref[...], t, (((0,), (0,)), ((), ())),
                        preferred_element_type=jnp.float32) + b1_ref[...]
    mu = jnp.mean(t, axis=1, keepdims=True)
    var = jnp.mean((t - mu) ** 2, axis=1, keepdims=True)
    t = (t - mu) * lax.rsqrt(var + 1e-5) * g_ref[...] + bb_ref[...]
    t = jnp.maximum(t, 0.0)
    h2 = lax.dot_general(w2_ref[...], t, (((0,), (0,)), ((), ())),
                         preferred_element_type=jnp.float32) + b2_ref[...]
    out_ref[...] = jnp.maximum(h2, 0.0)


def _mlp(h_t, agg_t, w1, b1, g, bb, w2, b2):
    return pl.pallas_call(
        _mlp_body,
        out_shape=jax.ShapeDtypeStruct((_H, _N), jnp.float32),
    )(h_t, agg_t, w1, b1.reshape(_H, 1), g.reshape(_H, 1), bb.reshape(_H, 1),
      w2, b2.reshape(_H, 1))


def _pool_body(h_ref, batch_ref, fcw_ref, out_ref):
    gids = lax.broadcasted_iota(jnp.int32, (_G, _N), 0)
    onehot = jnp.where(gids == batch_ref[...], 1.0, 0.0)
    pooled = lax.dot_general(onehot, h_ref[...], (((1,), (1,)), ((), ())),
                             preferred_element_type=jnp.float32)
    out_ref[...] = lax.dot_general(pooled, fcw_ref[...],
                                   (((1,), (0,)), ((), ())),
                                   preferred_element_type=jnp.float32)


def _pool(h_t, batch, fc_w, fc_b):
    out = pl.pallas_call(
        _pool_body,
        out_shape=jax.ShapeDtypeStruct((_G, 1), jnp.float32),
    )(h_t, batch.reshape(1, _N), fc_w)
    return out + fc_b


def kernel(x, edge_index, batch, edge_attr, node_W, node_b, edge_W, edge_b,
           lin1_W, lin1_b, bn_g, bn_b, lin2_W, lin2_b, fc_W, fc_b):
    src = edge_index[0].astype(jnp.int32)
    dst = edge_index[1].astype(jnp.int32)
    h_t = _node_enc(x, node_W, node_b)
    e_t = _edge_enc(edge_attr, edge_W, edge_b)
    import os as _os
    _dbg = _os.environ.get("DBG_XLA_MP", "0") == "1"
    for i in range(_LAYERS):
        if _dbg:
            msg = jax.nn.relu(h_t.T[src] + e_t.T)
            agg_t = jax.ops.segment_sum(msg, dst, num_segments=_N).T
        else:
            agg_t = _sc_message_pass(h_t, e_t, src, dst)
        h_t = _mlp(h_t, agg_t, lin1_W[i], lin1_b[i], bn_g[i], bn_b[i],
                   lin2_W[i], lin2_b[i])
    return _pool(h_t, batch.astype(jnp.int32), fc_W, fc_b)


# speedup vs baseline: 1.1048x; 1.1048x over previous
import functools

import jax
import jax.numpy as jnp
from jax import lax
from jax.experimental import pallas as pl
from jax.experimental.pallas import tpu as pltpu
from jax.experimental.pallas import tpu_sc as plsc

_N = 10000
_E = 320000
_H = 128
_G = 64
_LAYERS = 3

_CHUNK = 3200
_NCH = _E // _CHUNK
_FPW = 4


def _sc_message_pass(h_t, e_t, src, dst):
    mesh = plsc.VectorSubcoreMesh(core_axis_name="c", subcore_axis_name="s")

    @functools.partial(
        pl.kernel,
        mesh=mesh,
        out_type=jax.ShapeDtypeStruct((_H, _N), jnp.float32),
        scratch_types=(
            [pltpu.VMEM((_N,), jnp.float32) for _ in range(_FPW)]
            + [pltpu.VMEM((_N,), jnp.float32) for _ in range(_FPW)]
            + [
                pltpu.VMEM((2, _CHUNK), jnp.int32),
                pltpu.VMEM((2, _CHUNK), jnp.int32),
                pltpu.VMEM((2, _FPW // 2, _CHUNK), jnp.int32),
            ]
            + [pltpu.SemaphoreType.DMA for _ in range(6)]
        ),
        compiler_params=pltpu.CompilerParams(needs_layout_passes=False),
    )
    def mp_kernel(h_hbm, e_hbm, src_hbm, dst_hbm, agg_hbm,
                  h0, h1, h2, h3, a0, a1, a2, a3,
                  src_v, dst_v, e_v,
                  ss0, ss1, sd0, sd1, se0, se1):
        h_f = [h0, h1, h2, h3]
        a_f = [a0, a1, a2, a3]
        sem_s = [ss0, ss1]
        sem_d = [sd0, sd1]
        sem_e = [se0, se1]

        wid = lax.axis_index("s") * 2 + lax.axis_index("c")
        f0 = wid * _FPW

        for f in range(_FPW):
            pltpu.sync_copy(h_hbm.at[f0 + f], h_f[f])

        def zero_body(i, _):
            ii = pl.multiple_of(i * 16, 16)
            z = jnp.zeros((16,), jnp.float32)
            for f in range(_FPW):
                a_f[f][pl.ds(ii, 16)] = z
            return _

        lax.fori_loop(0, _N // 16, zero_body, None)

        def start_chunk(c, b):
            off = pl.multiple_of(c * _CHUNK, _CHUNK)
            pltpu.async_copy(src_hbm.at[pl.ds(off, _CHUNK)], src_v.at[b],
                             sem_s[b])
            pltpu.async_copy(dst_hbm.at[pl.ds(off, _CHUNK)], dst_v.at[b],
                             sem_d[b])
            pltpu.async_copy(
                e_hbm.at[pl.ds(wid * (_FPW // 2), _FPW // 2),
                         pl.ds(off, _CHUNK)],
                e_v.at[b], sem_e[b])

        def wait_chunk(b):
            pltpu.make_async_copy(src_hbm.at[pl.ds(0, _CHUNK)], src_v.at[b],
                                  sem_s[b]).wait()
            pltpu.make_async_copy(dst_hbm.at[pl.ds(0, _CHUNK)], dst_v.at[b],
                                  sem_d[b]).wait()
            pltpu.make_async_copy(
                e_hbm.at[pl.ds(0, _FPW // 2), pl.ds(0, _CHUNK)],
                e_v.at[b], sem_e[b]).wait()

        start_chunk(0, 0)
        start_chunk(1, 1)

        def edge_body(j, b):
            jj = pl.multiple_of(j * 16, 16)
            sv = src_v[b, pl.ds(jj, 16)]
            dv = dst_v[b, pl.ds(jj, 16)]
            for p in range(_FPW // 2):
                w = e_v[b, p, pl.ds(jj, 16)]
                ev_lo = plsc.bitcast(lax.shift_left(w, 16), jnp.float32)
                ev_hi = plsc.bitcast(
                    lax.bitwise_and(w, jnp.int32(-65536)), jnp.float32)
                for q, ev in ((0, ev_lo), (1, ev_hi)):
                    f = p * 2 + q
                    hv = plsc.load_gather(h_f[f], [sv])
                    m = jnp.maximum(hv + ev, 0.0)
                    plsc.addupdate_scatter(a_f[f], [dv], m)

        def outer_body(g, _):
            for b in range(2):
                c = g * 2 + b
                wait_chunk(b)

                @plsc.parallel_loop(0, _CHUNK // 16, unroll=8)
                def _(j, _b=b):
                    edge_body(j, _b)

                @pl.when(c + 2 < _NCH)
                def _():
                    start_chunk(c + 2, b)
            return _

        lax.fori_loop(0, _NCH // 2, outer_body, None)

        for f in range(_FPW):
            pltpu.sync_copy(a_f[f], agg_hbm.at[f0 + f])

    return mp_kernel(h_t, e_t, src, dst)


def _node_enc_body(x_ref, w_ref, b_ref, out_ref):
    out_ref[...] = lax.dot_general(
        w_ref[...], x_ref[...], (((0,), (1,)), ((), ())),
        preferred_element_type=jnp.float32) + b_ref[...]


def _node_enc(x, w, b):
    return pl.pallas_call(
        _node_enc_body,
        out_shape=jax.ShapeDtypeStruct((_H, _N), jnp.float32),
    )(x, w, b.reshape(_H, 1))


def _rne_bf16_bits(x):
    u = lax.bitcast_convert_type(x, jnp.uint32)
    return (u + jnp.uint32(0x7FFF)
            + (lax.shift_right_logical(u, jnp.uint32(16)) & jnp.uint32(1))) >> jnp.uint32(16)


def _edge_enc_body(attr_ref, w_ref, b_ref, out_hbm, scratch, sem, *, block):
    i = pl.program_id(0)
    x = lax.dot_general(
        w_ref[...], attr_ref[...], (((0,), (1,)), ((), ())),
        preferred_element_type=jnp.float32) + b_ref[...]
    lo = _rne_bf16_bits(x[0:_H // 2, :])
    hi = _rne_bf16_bits(x[_H // 2:, :])
    scratch[...] = lax.bitcast_convert_type(
        lax.shift_left(hi, jnp.uint32(16)) | lo, jnp.int32)
    pltpu.async_copy(scratch, out_hbm.at[:, pl.ds(i * block, block)],
                     sem).wait()


def _edge_enc(attr, w, b, block=16000):
    return pl.pallas_call(
        functools.partial(_edge_enc_body, block=block),
        grid=(_E // block,),
        in_specs=[
            pl.BlockSpec((block, 16), lambda i: (i, 0)),
            pl.BlockSpec((16, _H), lambda i: (0, 0)),
            pl.BlockSpec((_H, 1), lambda i: (0, 0)),
        ],
        out_specs=pl.BlockSpec(memory_space=pl.ANY),
        scratch_shapes=[pltpu.VMEM((_H // 2, block), jnp.int32),
                        pltpu.SemaphoreType.DMA],
        out_shape=jax.ShapeDtypeStruct((_H // 2, _E), jnp.int32),
    )(attr, w, b.reshape(_H, 1))


def _mlp_body(h_ref, agg_ref, w1_ref, b1_ref, g_ref, bb_ref, w2_ref, b2_ref,
              out_ref):
    t = h_ref[...] + agg_ref[...]
    t = lax.dot_general(w1_ref[...], t, (((0,), (0,)), ((), ())),
                        preferred_element_type=jnp.float32) + b1_ref[...]
    mu = jnp.mean(t, axis=1, keepdims=True)
    var = jnp.mean((t - mu) ** 2, axis=1, keepdims=True)
    t = (t - mu) * lax.rsqrt(var + 1e-5) * g_ref[...] + bb_ref[...]
    t = jnp.maximum(t, 0.0)
    h2 = lax.dot_general(w2_ref[...], t, (((0,), (0,)), ((), ())),
                         preferred_element_type=jnp.float32) + b2_ref[...]
    out_ref[...] = jnp.maximum(h2, 0.0)


def _mlp(h_t, agg_t, w1, b1, g, bb, w2, b2):
    return pl.pallas_call(
        _mlp_body,
        out_shape=jax.ShapeDtypeStruct((_H, _N), jnp.float32),
    )(h_t, agg_t, w1, b1.reshape(_H, 1), g.reshape(_H, 1), bb.reshape(_H, 1),
      w2, b2.reshape(_H, 1))


def _pool_body(h_ref, batch_ref, fcw_ref, out_ref):
    gids = lax.broadcasted_iota(jnp.int32, (_G, _N), 0)
    onehot = jnp.where(gids == batch_ref[...], 1.0, 0.0)
    pooled = lax.dot_general(onehot, h_ref[...], (((1,), (1,)), ((), ())),
                             preferred_element_type=jnp.float32)
    out_ref[...] = lax.dot_general(pooled, fcw_ref[...],
                                   (((1,), (0,)), ((), ())),
                                   preferred_element_type=jnp.float32)


def _pool(h_t, batch, fc_w, fc_b):
    out = pl.pallas_call(
        _pool_body,
        out_shape=jax.ShapeDtypeStruct((_G, 1), jnp.float32),
    )(h_t, batch.reshape(1, _N), fc_w)
    return out + fc_b


def kernel(x, edge_index, batch, edge_attr, node_W, node_b, edge_W, edge_b,
           lin1_W, lin1_b, bn_g, bn_b, lin2_W, lin2_b, fc_W, fc_b):
    src = edge_index[0].astype(jnp.int32)
    dst = edge_index[1].astype(jnp.int32)
    h_t = _node_enc(x, node_W, node_b)
    ew_perm = jnp.concatenate([edge_W[:, 0::2], edge_W[:, 1::2]], axis=1)
    eb_perm = jnp.concatenate([edge_b[0::2], edge_b[1::2]])
    e_t = _edge_enc(edge_attr, ew_perm, eb_perm)
    import os as _os
    _dbg = _os.environ.get("DBG_XLA_MP", "0") == "1"
    for i in range(_LAYERS):
        if _dbg:
            msg = jax.nn.relu(h_t.T[src] + e_t.T)
            agg_t = jax.ops.segment_sum(msg, dst, num_segments=_N).T
        else:
            agg_t = _sc_message_pass(h_t, e_t, src, dst)
        h_t = _mlp(h_t, agg_t, lin1_W[i], lin1_b[i], bn_g[i], bn_b[i],
                   lin2_W[i], lin2_b[i])
    return _pool(h_t, batch.astype(jnp.int32), fc_W, fc_b)


# speedup vs baseline: 1.2187x; 1.1031x over previous
import functools

import jax
import jax.numpy as jnp
from jax import lax
from jax.experimental import pallas as pl
from jax.experimental.pallas import tpu as pltpu
from jax.experimental.pallas import tpu_sc as plsc

_N = 10000
_E = 320000
_H = 128
_G = 64
_LAYERS = 3

_CHUNK = 3200
_NCH = _E // _CHUNK
_FPW = 4


def _sc_message_pass(h_t, e_t, src, dst):
    mesh = plsc.VectorSubcoreMesh(core_axis_name="c", subcore_axis_name="s")

    @functools.partial(
        pl.kernel,
        mesh=mesh,
        out_type=jax.ShapeDtypeStruct((_H, _N), jnp.float32),
        scratch_types=(
            [pltpu.VMEM((_N,), jnp.float32) for _ in range(_FPW)]
            + [pltpu.VMEM((_N,), jnp.float32) for _ in range(_FPW)]
            + [
                pltpu.VMEM((2, _CHUNK), jnp.int32),
                pltpu.VMEM((2, _CHUNK), jnp.int32),
                pltpu.VMEM((2, _FPW // 2, _CHUNK), jnp.int32),
            ]
            + [pltpu.SemaphoreType.DMA for _ in range(6)]
        ),
        compiler_params=pltpu.CompilerParams(needs_layout_passes=False),
    )
    def mp_kernel(h_hbm, e_hbm, src_hbm, dst_hbm, agg_hbm,
                  h0, h1, h2, h3, a0, a1, a2, a3,
                  src_v, dst_v, e_v,
                  ss0, ss1, sd0, sd1, se0, se1):
        h_f = [h0, h1, h2, h3]
        a_f = [a0, a1, a2, a3]
        sem_s = [ss0, ss1]
        sem_d = [sd0, sd1]
        sem_e = [se0, se1]

        wid = lax.axis_index("s") * 2 + lax.axis_index("c")
        f0 = wid * _FPW

        for f in range(_FPW):
            pltpu.sync_copy(h_hbm.at[f0 + f], h_f[f])

        def zero_body(i, _):
            ii = pl.multiple_of(i * 16, 16)
            z = jnp.zeros((16,), jnp.float32)
            for f in range(_FPW):
                a_f[f][pl.ds(ii, 16)] = z
            return _

        lax.fori_loop(0, _N // 16, zero_body, None)

        def start_chunk(c, b):
            off = pl.multiple_of(c * _CHUNK, _CHUNK)
            pltpu.async_copy(src_hbm.at[pl.ds(off, _CHUNK)], src_v.at[b],
                             sem_s[b])
            pltpu.async_copy(dst_hbm.at[pl.ds(off, _CHUNK)], dst_v.at[b],
                             sem_d[b])
            pltpu.async_copy(
                e_hbm.at[pl.ds(wid * (_FPW // 2), _FPW // 2),
                         pl.ds(off, _CHUNK)],
                e_v.at[b], sem_e[b])

        def wait_chunk(b):
            pltpu.make_async_copy(src_hbm.at[pl.ds(0, _CHUNK)], src_v.at[b],
                                  sem_s[b]).wait()
            pltpu.make_async_copy(dst_hbm.at[pl.ds(0, _CHUNK)], dst_v.at[b],
                                  sem_d[b]).wait()
            pltpu.make_async_copy(
                e_hbm.at[pl.ds(0, _FPW // 2), pl.ds(0, _CHUNK)],
                e_v.at[b], sem_e[b]).wait()

        start_chunk(0, 0)
        start_chunk(1, 1)

        def edge_body(j, b):
            jj = pl.multiple_of(j * 16, 16)
            sv = src_v[b, pl.ds(jj, 16)]
            dv = dst_v[b, pl.ds(jj, 16)]
            for p in range(_FPW // 2):
                w = e_v[b, p, pl.ds(jj, 16)]
                ev_lo = plsc.bitcast(lax.shift_left(w, 16), jnp.float32)
                ev_hi = plsc.bitcast(
                    lax.bitwise_and(w, jnp.int32(-65536)), jnp.float32)
                for q, ev in ((0, ev_lo), (1, ev_hi)):
                    f = p * 2 + q
                    hv = plsc.load_gather(h_f[f], [sv])
                    m = jnp.maximum(hv + ev, 0.0)
                    plsc.addupdate_scatter(a_f[f], [dv], m)

        def outer_body(g, _):
            for b in range(2):
                c = g * 2 + b
                wait_chunk(b)

                @plsc.parallel_loop(0, _CHUNK // 16, unroll=8)
                def _(j, _b=b):
                    edge_body(j, _b)

                @pl.when(c + 2 < _NCH)
                def _():
                    start_chunk(c + 2, b)
            return _

        lax.fori_loop(0, _NCH // 2, outer_body, None)

        for f in range(_FPW):
            pltpu.sync_copy(a_f[f], agg_hbm.at[f0 + f])

    return mp_kernel(h_t, e_t, src, dst)


def _node_enc_body(x_ref, w_ref, b_ref, out_ref):
    out_ref[...] = lax.dot_general(
        w_ref[...], x_ref[...], (((0,), (1,)), ((), ())),
        preferred_element_type=jnp.float32) + b_ref[...]


def _node_enc(x, w, b):
    return pl.pallas_call(
        _node_enc_body,
        out_shape=jax.ShapeDtypeStruct((_H, _N), jnp.float32),
    )(x, w, b.reshape(_H, 1))


def _rne_bf16_bits(x):
    u = lax.bitcast_convert_type(x, jnp.uint32)
    return (u + jnp.uint32(0x7FFF)
            + (lax.shift_right_logical(u, jnp.uint32(16)) & jnp.uint32(1))) >> jnp.uint32(16)


def _edge_enc_body(attr_ref, w_ref, b_ref, out_hbm, scratch, sem, *, block):
    i = pl.program_id(0)
    x = lax.dot_general(
        w_ref[...], attr_ref[...], (((0,), (0,)), ((), ())),
        preferred_element_type=jnp.float32) + b_ref[...]
    lo = _rne_bf16_bits(x[0:_H // 2, :])
    hi = _rne_bf16_bits(x[_H // 2:, :])
    scratch[...] = lax.bitcast_convert_type(
        lax.shift_left(hi, jnp.uint32(16)) | lo, jnp.int32)
    pltpu.async_copy(scratch, out_hbm.at[:, pl.ds(i * block, block)],
                     sem).wait()


def _edge_enc(attr, w, b, block=16000):
    return pl.pallas_call(
        functools.partial(_edge_enc_body, block=block),
        grid=(_E // block,),
        in_specs=[
            pl.BlockSpec((16, block), lambda i: (0, i)),
            pl.BlockSpec((16, _H), lambda i: (0, 0)),
            pl.BlockSpec((_H, 1), lambda i: (0, 0)),
        ],
        out_specs=pl.BlockSpec(memory_space=pl.ANY),
        scratch_shapes=[pltpu.VMEM((_H // 2, block), jnp.int32),
                        pltpu.SemaphoreType.DMA],
        out_shape=jax.ShapeDtypeStruct((_H // 2, _E), jnp.int32),
    )(attr, w, b.reshape(_H, 1))


def _mlp_body(h_ref, agg_ref, w1_ref, b1_ref, g_ref, bb_ref, w2_ref, b2_ref,
              out_ref):
    t = h_ref[...] + agg_ref[...]
    t = lax.dot_general(w1_ref[...], t, (((0,), (0,)), ((), ())),
                        preferred_element_type=jnp.float32) + b1_ref[...]
    mu = jnp.mean(t, axis=1, keepdims=True)
    var = jnp.mean((t - mu) ** 2, axis=1, keepdims=True)
    t = (t - mu) * lax.rsqrt(var + 1e-5) * g_ref[...] + bb_ref[...]
    t = jnp.maximum(t, 0.0)
    h2 = lax.dot_general(w2_ref[...], t, (((0,), (0,)), ((), ())),
                         preferred_element_type=jnp.float32) + b2_ref[...]
    out_ref[...] = jnp.maximum(h2, 0.0)


def _mlp(h_t, agg_t, w1, b1, g, bb, w2, b2):
    return pl.pallas_call(
        _mlp_body,
        out_shape=jax.ShapeDtypeStruct((_H, _N), jnp.float32),
    )(h_t, agg_t, w1, b1.reshape(_H, 1), g.reshape(_H, 1), bb.reshape(_H, 1),
      w2, b2.reshape(_H, 1))


def _pool_body(h_ref, batch_ref, fcw_ref, out_ref):
    gids = lax.broadcasted_iota(jnp.int32, (_G, _N), 0)
    onehot = jnp.where(gids == batch_ref[...], 1.0, 0.0)
    pooled = lax.dot_general(onehot, h_ref[...], (((1,), (1,)), ((), ())),
                             preferred_element_type=jnp.float32)
    out_ref[...] = lax.dot_general(pooled, fcw_ref[...],
                                   (((1,), (0,)), ((), ())),
                                   preferred_element_type=jnp.float32)


def _pool(h_t, batch, fc_w, fc_b):
    out = pl.pallas_call(
        _pool_body,
        out_shape=jax.ShapeDtypeStruct((_G, 1), jnp.float32),
    )(h_t, batch.reshape(1, _N), fc_w)
    return out + fc_b


def kernel(x, edge_index, batch, edge_attr, node_W, node_b, edge_W, edge_b,
           lin1_W, lin1_b, bn_g, bn_b, lin2_W, lin2_b, fc_W, fc_b):
    src = edge_index[0].astype(jnp.int32)
    dst = edge_index[1].astype(jnp.int32)
    h_t = _node_enc(x, node_W, node_b)
    ew_perm = jnp.concatenate([edge_W[:, 0::2], edge_W[:, 1::2]], axis=1)
    eb_perm = jnp.concatenate([edge_b[0::2], edge_b[1::2]])
    e_t = _edge_enc(edge_attr.T, ew_perm, eb_perm)
    import os as _os
    _dbg = _os.environ.get("DBG_XLA_MP", "0") == "1"
    for i in range(_LAYERS):
        if _dbg:
            msg = jax.nn.relu(h_t.T[src] + e_t.T)
            agg_t = jax.ops.segment_sum(msg, dst, num_segments=_N).T
        else:
            agg_t = _sc_message_pass(h_t, e_t, src, dst)
        h_t = _mlp(h_t, agg_t, lin1_W[i], lin1_b[i], bn_g[i], bn_b[i],
                   lin2_W[i], lin2_b[i])
    return _pool(h_t, batch.astype(jnp.int32), fc_W, fc_b)


# speedup vs baseline: 1.2751x; 1.0463x over previous
import functools

import jax
import jax.numpy as jnp
from jax import lax
from jax.experimental import pallas as pl
from jax.experimental.pallas import tpu as pltpu
from jax.experimental.pallas import tpu_sc as plsc

_N = 10000
_E = 320000
_H = 128
_G = 64
_LAYERS = 3

_CHUNK = 3200
_NCH = _E // _CHUNK
_FPW = 4


def _sc_message_pass(h_t, e_t, src, dst):
    mesh = plsc.VectorSubcoreMesh(core_axis_name="c", subcore_axis_name="s")

    @functools.partial(
        pl.kernel,
        mesh=mesh,
        out_type=jax.ShapeDtypeStruct((_H, _N), jnp.float32),
        scratch_types=(
            [pltpu.VMEM((_N,), jnp.float32) for _ in range(_FPW)]
            + [pltpu.VMEM((_N,), jnp.float32) for _ in range(_FPW)]
            + [
                pltpu.VMEM((2, _CHUNK), jnp.int32),
                pltpu.VMEM((2, _CHUNK), jnp.int32),
                pltpu.VMEM((2, _FPW // 2, _CHUNK), jnp.int32),
            ]
            + [pltpu.SemaphoreType.DMA for _ in range(6)]
        ),
        compiler_params=pltpu.CompilerParams(needs_layout_passes=False),
    )
    def mp_kernel(h_hbm, e_hbm, src_hbm, dst_hbm, agg_hbm,
                  h0, h1, h2, h3, a0, a1, a2, a3,
                  src_v, dst_v, e_v,
                  ss0, ss1, sd0, sd1, se0, se1):
        h_f = [h0, h1, h2, h3]
        a_f = [a0, a1, a2, a3]
        sem_s = [ss0, ss1]
        sem_d = [sd0, sd1]
        sem_e = [se0, se1]

        wid = lax.axis_index("s") * 2 + lax.axis_index("c")
        f0 = wid * _FPW

        for f in range(_FPW):
            pltpu.sync_copy(h_hbm.at[f0 + f], h_f[f])

        def zero_body(i, _):
            ii = pl.multiple_of(i * 16, 16)
            z = jnp.zeros((16,), jnp.float32)
            for f in range(_FPW):
                a_f[f][pl.ds(ii, 16)] = z
            return _

        lax.fori_loop(0, _N // 16, zero_body, None)

        def start_chunk(c, b):
            off = pl.multiple_of(c * _CHUNK, _CHUNK)
            pltpu.async_copy(src_hbm.at[pl.ds(off, _CHUNK)], src_v.at[b],
                             sem_s[b])
            pltpu.async_copy(dst_hbm.at[pl.ds(off, _CHUNK)], dst_v.at[b],
                             sem_d[b])
            pltpu.async_copy(
                e_hbm.at[pl.ds(wid * (_FPW // 2), _FPW // 2),
                         pl.ds(off, _CHUNK)],
                e_v.at[b], sem_e[b])

        def wait_chunk(b):
            pltpu.make_async_copy(src_hbm.at[pl.ds(0, _CHUNK)], src_v.at[b],
                                  sem_s[b]).wait()
            pltpu.make_async_copy(dst_hbm.at[pl.ds(0, _CHUNK)], dst_v.at[b],
                                  sem_d[b]).wait()
            pltpu.make_async_copy(
                e_hbm.at[pl.ds(0, _FPW // 2), pl.ds(0, _CHUNK)],
                e_v.at[b], sem_e[b]).wait()

        start_chunk(0, 0)
        start_chunk(1, 1)

        def edge_body(j, b):
            jj = pl.multiple_of(j * 16, 16)
            sv = src_v[b, pl.ds(jj, 16)]
            dv = dst_v[b, pl.ds(jj, 16)]
            for p in range(_FPW // 2):
                w = e_v[b, p, pl.ds(jj, 16)]
                ev_lo = plsc.bitcast(lax.shift_left(w, 16), jnp.float32)
                ev_hi = plsc.bitcast(
                    lax.bitwise_and(w, jnp.int32(-65536)), jnp.float32)
                for q, ev in ((0, ev_lo), (1, ev_hi)):
                    f = p * 2 + q
                    hv = plsc.load_gather(h_f[f], [sv])
                    m = jnp.maximum(hv + ev, 0.0)
                    plsc.addupdate_scatter(a_f[f], [dv], m)

        def outer_body(g, _):
            for b in range(2):
                c = g * 2 + b
                wait_chunk(b)

                @plsc.parallel_loop(0, _CHUNK // 16, unroll=8)
                def _(j, _b=b):
                    edge_body(j, _b)

                @pl.when(c + 2 < _NCH)
                def _():
                    start_chunk(c + 2, b)
            return _

        lax.fori_loop(0, _NCH // 2, outer_body, None)

        for f in range(_FPW):
            pltpu.sync_copy(a_f[f], agg_hbm.at[f0 + f])

    return mp_kernel(h_t, e_t, src, dst)


def _node_enc_body(x_ref, w_ref, b_ref, out_ref):
    out_ref[...] = lax.dot_general(
        w_ref[...], x_ref[...], (((0,), (1,)), ((), ())),
        preferred_element_type=jnp.float32) + b_ref[...]


def _node_enc(x, w, b):
    return pl.pallas_call(
        _node_enc_body,
        out_shape=jax.ShapeDtypeStruct((_H, _N), jnp.float32),
    )(x, w, b.reshape(_H, 1))


def _edge_enc_body(attr_ref, w_ref, b_ref, out_hbm, s0, s1, sem0, sem1,
                   *, block, nstep):
    i = pl.program_id(0)
    x = lax.dot_general(
        w_ref[...], attr_ref[...], (((0,), (0,)), ((), ())),
        preferred_element_type=jnp.float32) + b_ref[...]
    u_lo = lax.bitcast_convert_type(x[0:_H // 2, :], jnp.uint32)
    u_hi = lax.bitcast_convert_type(x[_H // 2:, :], jnp.uint32)
    word = lax.bitcast_convert_type(
        ((u_hi + jnp.uint32(0x8000)) & jnp.uint32(0xFFFF0000))
        | ((u_lo + jnp.uint32(0x8000)) >> jnp.uint32(16)), jnp.int32)
    for par, (s, sem) in enumerate(((s0, sem0), (s1, sem1))):
        @pl.when(lax.rem(i, 2) == par)
        def _():
            @pl.when(i >= 2)
            def _():
                pltpu.make_async_copy(
                    s, out_hbm.at[:, pl.ds(0, block)], sem).wait()
            s[...] = word
            cp = pltpu.async_copy(s, out_hbm.at[:, pl.ds(i * block, block)],
                                  sem)

            @pl.when(i == nstep - 1)
            def _():
                cp.wait()

    @pl.when(i == nstep - 1)
    def _():
        for par, (s, sem) in enumerate(((s0, sem0), (s1, sem1))):
            @pl.when(lax.rem(i, 2) != par)
            def _():
                pltpu.make_async_copy(
                    s, out_hbm.at[:, pl.ds(0, block)], sem).wait()


def _edge_enc(attr, w, b, block=16000):
    nstep = _E // block
    return pl.pallas_call(
        functools.partial(_edge_enc_body, block=block, nstep=nstep),
        grid=(nstep,),
        in_specs=[
            pl.BlockSpec((16, block), lambda i: (0, i)),
            pl.BlockSpec((16, _H), lambda i: (0, 0)),
            pl.BlockSpec((_H, 1), lambda i: (0, 0)),
        ],
        out_specs=pl.BlockSpec(memory_space=pl.ANY),
        scratch_shapes=[pltpu.VMEM((_H // 2, block), jnp.int32),
                        pltpu.VMEM((_H // 2, block), jnp.int32),
                        pltpu.SemaphoreType.DMA, pltpu.SemaphoreType.DMA],
        out_shape=jax.ShapeDtypeStruct((_H // 2, _E), jnp.int32),
    )(attr, w, b.reshape(_H, 1))


def _mlp_body(h_ref, agg_ref, w1_ref, b1_ref, g_ref, bb_ref, w2_ref, b2_ref,
              out_ref):
    t = h_ref[...] + agg_ref[...]
    t = lax.dot_general(w1_ref[...], t, (((0,), (0,)), ((), ())),
                        preferred_element_type=jnp.float32) + b1_ref[...]
    mu = jnp.mean(t, axis=1, keepdims=True)
    var = jnp.mean((t - mu) ** 2, axis=1, keepdims=True)
    t = (t - mu) * lax.rsqrt(var + 1e-5) * g_ref[...] + bb_ref[...]
    t = jnp.maximum(t, 0.0)
    h2 = lax.dot_general(w2_ref[...], t, (((0,), (0,)), ((), ())),
                         preferred_element_type=jnp.float32) + b2_ref[...]
    out_ref[...] = jnp.maximum(h2, 0.0)


def _mlp(h_t, agg_t, w1, b1, g, bb, w2, b2):
    return pl.pallas_call(
        _mlp_body,
        out_shape=jax.ShapeDtypeStruct((_H, _N), jnp.float32),
    )(h_t, agg_t, w1, b1.reshape(_H, 1), g.reshape(_H, 1), bb.reshape(_H, 1),
      w2, b2.reshape(_H, 1))


def _pool_body(h_ref, batch_ref, fcw_ref, out_ref):
    gids = lax.broadcasted_iota(jnp.int32, (_G, _N), 0)
    onehot = jnp.where(gids == batch_ref[...], 1.0, 0.0)
    pooled = lax.dot_general(onehot, h_ref[...], (((1,), (1,)), ((), ())),
                             preferred_element_type=jnp.float32)
    out_ref[...] = lax.dot_general(pooled, fcw_ref[...],
                                   (((1,), (0,)), ((), ())),
                                   preferred_element_type=jnp.float32)


def _pool(h_t, batch, fc_w, fc_b):
    out = pl.pallas_call(
        _pool_body,
        out_shape=jax.ShapeDtypeStruct((_G, 1), jnp.float32),
    )(h_t, batch.reshape(1, _N), fc_w)
    return out + fc_b


def kernel(x, edge_index, batch, edge_attr, node_W, node_b, edge_W, edge_b,
           lin1_W, lin1_b, bn_g, bn_b, lin2_W, lin2_b, fc_W, fc_b):
    src = edge_index[0].astype(jnp.int32)
    dst = edge_index[1].astype(jnp.int32)
    h_t = _node_enc(x, node_W, node_b)
    ew_perm = jnp.concatenate([edge_W[:, 0::2], edge_W[:, 1::2]], axis=1)
    eb_perm = jnp.concatenate([edge_b[0::2], edge_b[1::2]])
    e_t = _edge_enc(edge_attr.T, ew_perm, eb_perm)
    import os as _os
    _dbg = _os.environ.get("DBG_XLA_MP", "0") == "1"
    for i in range(_LAYERS):
        if _dbg:
            msg = jax.nn.relu(h_t.T[src] + e_t.T)
            agg_t = jax.ops.segment_sum(msg, dst, num_segments=_N).T
        else:
            agg_t = _sc_message_pass(h_t, e_t, src, dst)
        h_t = _mlp(h_t, agg_t, lin1_W[i], lin1_b[i], bn_g[i], bn_b[i],
                   lin2_W[i], lin2_b[i])
    return _pool(h_t, batch.astype(jnp.int32), fc_W, fc_b)


# speedup vs baseline: 1.4028x; 1.1002x over previous
import functools

import jax
import jax.numpy as jnp
from jax import lax
from jax.experimental import pallas as pl
from jax.experimental.pallas import tpu as pltpu
from jax.experimental.pallas import tpu_sc as plsc

_N = 10000
_E = 320000
_H = 128
_G = 64
_LAYERS = 3

_CHUNK = 3200
_NCH = _E // _CHUNK
_FPW = 4


def _sc_message_pass(h_t, e_t, src_dst):
    mesh = plsc.VectorSubcoreMesh(core_axis_name="c", subcore_axis_name="s")

    @functools.partial(
        pl.kernel,
        mesh=mesh,
        out_type=jax.ShapeDtypeStruct((_H, _N), jnp.float32),
        scratch_types=(
            [pltpu.VMEM((_N,), jnp.float32) for _ in range(_FPW)]
            + [pltpu.VMEM((_N,), jnp.float32) for _ in range(_FPW)]
            + [
                pltpu.VMEM((2, _CHUNK), jnp.int32),
                pltpu.VMEM((2, _FPW // 2, _CHUNK), jnp.int32),
            ]
            + [pltpu.SemaphoreType.DMA for _ in range(4)]
        ),
        compiler_params=pltpu.CompilerParams(needs_layout_passes=False),
    )
    def mp_kernel(h_hbm, e_hbm, sd_hbm, agg_hbm,
                  h0, h1, h2, h3, a0, a1, a2, a3,
                  sd_v, e_v,
                  ss0, ss1, se0, se1):
        h_f = [h0, h1, h2, h3]
        a_f = [a0, a1, a2, a3]
        sem_s = [ss0, ss1]
        sem_e = [se0, se1]

        wid = lax.axis_index("s") * 2 + lax.axis_index("c")
        f0 = wid * _FPW

        for f in range(_FPW):
            pltpu.sync_copy(h_hbm.at[f0 + f], h_f[f])

        def zero_body(i, _):
            ii = pl.multiple_of(i * 16, 16)
            z = jnp.zeros((16,), jnp.float32)
            for f in range(_FPW):
                a_f[f][pl.ds(ii, 16)] = z
            return _

        lax.fori_loop(0, _N // 16, zero_body, None)

        def start_chunk(c, b):
            off = pl.multiple_of(c * _CHUNK, _CHUNK)
            pltpu.async_copy(sd_hbm.at[pl.ds(off, _CHUNK)], sd_v.at[b],
                             sem_s[b])
            pltpu.async_copy(
                e_hbm.at[pl.ds(wid * (_FPW // 2), _FPW // 2),
                         pl.ds(off, _CHUNK)],
                e_v.at[b], sem_e[b])

        def wait_chunk(b):
            pltpu.make_async_copy(sd_hbm.at[pl.ds(0, _CHUNK)], sd_v.at[b],
                                  sem_s[b]).wait()
            pltpu.make_async_copy(
                e_hbm.at[pl.ds(0, _FPW // 2), pl.ds(0, _CHUNK)],
                e_v.at[b], sem_e[b]).wait()

        start_chunk(0, 0)
        start_chunk(1, 1)

        def edge_body(j, b):
            jj = pl.multiple_of(j * 16, 16)
            sd = sd_v[b, pl.ds(jj, 16)]
            sv = lax.bitwise_and(sd, jnp.int32(0xFFFF))
            dv = lax.shift_right_logical(sd, 16)
            for p in range(_FPW // 2):
                w = e_v[b, p, pl.ds(jj, 16)]
                ev_lo = plsc.bitcast(lax.shift_left(w, 16), jnp.float32)
                ev_hi = plsc.bitcast(
                    lax.bitwise_and(w, jnp.int32(-65536)), jnp.float32)
                for q, ev in ((0, ev_lo), (1, ev_hi)):
                    f = p * 2 + q
                    hv = plsc.load_gather(h_f[f], [sv])
                    m = jnp.maximum(hv + ev, 0.0)
                    plsc.addupdate_scatter(a_f[f], [dv], m)

        def outer_body(g, _):
            for b in range(2):
                c = g * 2 + b
                wait_chunk(b)

                @plsc.parallel_loop(0, _CHUNK // 16, unroll=8)
                def _(j, _b=b):
                    edge_body(j, _b)

                @pl.when(c + 2 < _NCH)
                def _():
                    start_chunk(c + 2, b)
            return _

        lax.fori_loop(0, _NCH // 2, outer_body, None)

        for f in range(_FPW):
            pltpu.sync_copy(a_f[f], agg_hbm.at[f0 + f])

    return mp_kernel(h_t, e_t, src_dst)


def _node_enc_body(x_ref, w_ref, b_ref, out_ref):
    out_ref[...] = lax.dot_general(
        w_ref[...], x_ref[...], (((0,), (1,)), ((), ())),
        preferred_element_type=jnp.float32) + b_ref[...]


def _node_enc(x, w, b):
    return pl.pallas_call(
        _node_enc_body,
        out_shape=jax.ShapeDtypeStruct((_H, _N), jnp.float32),
    )(x, w, b.reshape(_H, 1))


def _edge_enc_body(attr_ref, w_ref, b_ref, out_hbm, s0, s1, sem0, sem1,
                   *, block, nstep):
    i = pl.program_id(0)
    x = lax.dot_general(
        w_ref[...], attr_ref[...], (((0,), (0,)), ((), ())),
        preferred_element_type=jnp.float32) + b_ref[...]
    u_lo = lax.bitcast_convert_type(x[0:_H // 2, :], jnp.uint32)
    u_hi = lax.bitcast_convert_type(x[_H // 2:, :], jnp.uint32)
    word = lax.bitcast_convert_type(
        ((u_hi + jnp.uint32(0x8000)) & jnp.uint32(0xFFFF0000))
        | ((u_lo + jnp.uint32(0x8000)) >> jnp.uint32(16)), jnp.int32)
    for par, (s, sem) in enumerate(((s0, sem0), (s1, sem1))):
        @pl.when(lax.rem(i, 2) == par)
        def _():
            @pl.when(i >= 2)
            def _():
                pltpu.make_async_copy(
                    s, out_hbm.at[:, pl.ds(0, block)], sem).wait()
            s[...] = word
            cp = pltpu.async_copy(s, out_hbm.at[:, pl.ds(i * block, block)],
                                  sem)

            @pl.when(i == nstep - 1)
            def _():
                cp.wait()

    @pl.when(i == nstep - 1)
    def _():
        for par, (s, sem) in enumerate(((s0, sem0), (s1, sem1))):
            @pl.when(lax.rem(i, 2) != par)
            def _():
                pltpu.make_async_copy(
                    s, out_hbm.at[:, pl.ds(0, block)], sem).wait()


def _edge_enc(attr, w, b, block=16000):
    nstep = _E // block
    return pl.pallas_call(
        functools.partial(_edge_enc_body, block=block, nstep=nstep),
        grid=(nstep,),
        in_specs=[
            pl.BlockSpec((16, block), lambda i: (0, i)),
            pl.BlockSpec((16, _H), lambda i: (0, 0)),
            pl.BlockSpec((_H, 1), lambda i: (0, 0)),
        ],
        out_specs=pl.BlockSpec(memory_space=pl.ANY),
        scratch_shapes=[pltpu.VMEM((_H // 2, block), jnp.int32),
                        pltpu.VMEM((_H // 2, block), jnp.int32),
                        pltpu.SemaphoreType.DMA, pltpu.SemaphoreType.DMA],
        out_shape=jax.ShapeDtypeStruct((_H // 2, _E), jnp.int32),
    )(attr, w, b.reshape(_H, 1))


def _mlp_body(h_ref, agg_ref, w1_ref, b1_ref, g_ref, bb_ref, w2_ref, b2_ref,
              out_ref):
    t = h_ref[...] + agg_ref[...]
    t = lax.dot_general(w1_ref[...], t, (((0,), (0,)), ((), ())),
                        preferred_element_type=jnp.float32) + b1_ref[...]
    mu = jnp.mean(t, axis=1, keepdims=True)
    var = jnp.mean((t - mu) ** 2, axis=1, keepdims=True)
    t = (t - mu) * lax.rsqrt(var + 1e-5) * g_ref[...] + bb_ref[...]
    t = jnp.maximum(t, 0.0)
    h2 = lax.dot_general(w2_ref[...], t, (((0,), (0,)), ((), ())),
                         preferred_element_type=jnp.float32) + b2_ref[...]
    out_ref[...] = jnp.maximum(h2, 0.0)


def _mlp(h_t, agg_t, w1, b1, g, bb, w2, b2):
    return pl.pallas_call(
        _mlp_body,
        out_shape=jax.ShapeDtypeStruct((_H, _N), jnp.float32),
    )(h_t, agg_t, w1, b1.reshape(_H, 1), g.reshape(_H, 1), bb.reshape(_H, 1),
      w2, b2.reshape(_H, 1))


def _pool_body(h_ref, batch_ref, fcw_ref, out_ref):
    gids = lax.broadcasted_iota(jnp.int32, (_G, _N), 0)
    onehot = jnp.where(gids == batch_ref[...], 1.0, 0.0)
    pooled = lax.dot_general(onehot, h_ref[...], (((1,), (1,)), ((), ())),
                             preferred_element_type=jnp.float32)
    out_ref[...] = lax.dot_general(pooled, fcw_ref[...],
                                   (((1,), (0,)), ((), ())),
                                   preferred_element_type=jnp.float32)


def _pool(h_t, batch, fc_w, fc_b):
    out = pl.pallas_call(
        _pool_body,
        out_shape=jax.ShapeDtypeStruct((_G, 1), jnp.float32),
    )(h_t, batch.reshape(1, _N), fc_w)
    return out + fc_b


def kernel(x, edge_index, batch, edge_attr, node_W, node_b, edge_W, edge_b,
           lin1_W, lin1_b, bn_g, bn_b, lin2_W, lin2_b, fc_W, fc_b):
    src = edge_index[0].astype(jnp.int32)
    dst = edge_index[1].astype(jnp.int32)
    src_dst = src | lax.shift_left(dst, 16)
    h_t = _node_enc(x, node_W, node_b)
    ew_perm = jnp.concatenate([edge_W[:, 0::2], edge_W[:, 1::2]], axis=1)
    eb_perm = jnp.concatenate([edge_b[0::2], edge_b[1::2]])
    e_t = _edge_enc(edge_attr.T, ew_perm, eb_perm)
    import os as _os
    _dbg = _os.environ.get("DBG_XLA_MP", "0") == "1"
    for i in range(_LAYERS):
        if _dbg:
            msg = jax.nn.relu(h_t.T[src] + e_t.T)
            agg_t = jax.ops.segment_sum(msg, dst, num_segments=_N).T
        else:
            agg_t = _sc_message_pass(h_t, e_t, src_dst)
        h_t = _mlp(h_t, agg_t, lin1_W[i], lin1_b[i], bn_g[i], bn_b[i],
                   lin2_W[i], lin2_b[i])
    return _pool(h_t, batch.astype(jnp.int32), fc_W, fc_b)
